# Initial kernel scaffold; baseline (speedup 1.0000x reference)
import jax
import jax.numpy as jnp
from jax.experimental import pallas as pl


def kernel(x, edge_index, W_self1, b_self1, W_s2d1, b_s2d1, W_d2s1, b_d2s1, W_self2, b_self2, W_s2d2, b_s2d2, W_d2s2, b_d2s2):
    raise NotImplementedError("write your pallas kernel here")


# speedup vs baseline: 3.3678x; 3.3678x over previous
import functools

import jax
import jax.numpy as jnp
from jax import lax
from jax.experimental import pallas as pl
from jax.experimental.pallas import tpu as pltpu
from jax.experimental.pallas import tpu_sc as plsc

F32 = jnp.float32

N_NODES = 10000
DIM_IN = 256
DIM_HID = 512
DIM_OUT = 64
N_EDGES = 160000
ALPHA_MIX = 0.5

NC = 2
NS = 16

NP = 10240
EP = 163840
CB = 128
EDGES_PER_TILE = EP // NS
CHUNKS = EDGES_PER_TILE // CB
ROWS_PER_TILE = NP // NS
HALF = DIM_IN // 2

SELU_SCALE = 1.0507009873554805
SELU_ALPHA = 1.6732632423543772


def _sc_mesh():
    return plsc.VectorSubcoreMesh(
        core_axis_name="c", subcore_axis_name="s", num_cores=NC, num_subcores=NS
    )


def _k1_body(xa_h, xb_h, ei_h, zr_h, on_h,
             agg_a, agg_b, cnt_o,
             fidx, tidx, rows, onesv, acc, sem):
    c = lax.axis_index("c")
    s = lax.axis_index("s")
    r0 = s * ROWS_PER_TILE
    e0 = s * EDGES_PER_TILE

    pltpu.sync_copy(on_h, onesv)

    def scatter_pass(tbl_h, out_h):
        pltpu.sync_copy(zr_h, acc.at[pl.ds(r0, ROWS_PER_TILE)])
        plsc.subcore_barrier()

        def body(k, carry):
            base = e0 + k * CB
            pltpu.sync_copy(ei_h.at[1 - c, pl.ds(base, CB)], tidx)
            if tbl_h is None:
                pltpu.sync_copy(onesv, acc.at[tidx], add=True)
            else:
                pltpu.sync_copy(ei_h.at[c, pl.ds(base, CB)], fidx)
                pltpu.async_copy(tbl_h.at[fidx], rows, sem).wait()
                pltpu.sync_copy(rows, acc.at[tidx], add=True)
            return carry

        lax.fori_loop(0, CHUNKS, body, 0)
        plsc.subcore_barrier()
        pltpu.sync_copy(acc.at[pl.ds(r0, ROWS_PER_TILE)],
                        out_h.at[c, pl.ds(r0, ROWS_PER_TILE)])

    scatter_pass(None, cnt_o)
    scatter_pass(xa_h, agg_a)
    scatter_pass(xb_h, agg_b)


def _run_k1(xa, xb, ei, zrows, ones128):
    k1 = functools.partial(
        pl.kernel,
        out_type=[
            jax.ShapeDtypeStruct((NC, NP, HALF), F32),
            jax.ShapeDtypeStruct((NC, NP, HALF), F32),
            jax.ShapeDtypeStruct((NC, NP, HALF), F32),
        ],
        mesh=_sc_mesh(),
        scratch_types=[
            pltpu.VMEM((CB,), jnp.int32),
            pltpu.VMEM((CB,), jnp.int32),
            pltpu.VMEM((CB, HALF), F32),
            pltpu.VMEM((CB, HALF), F32),
            pltpu.VMEM_SHARED((NP, HALF), F32),
            pltpu.SemaphoreType.DMA,
        ],
    )(_k1_body)
    return k1(xa, xb, ei, zrows, ones128)


def _k3_body(ptbl_h, ei_h, zr_h, agg2_o, fidx, tidx, rows, acc, sem):
    c = lax.axis_index("c")
    s = lax.axis_index("s")
    r0 = s * ROWS_PER_TILE
    e0 = s * EDGES_PER_TILE

    pltpu.sync_copy(zr_h, acc.at[pl.ds(r0, ROWS_PER_TILE)])
    plsc.subcore_barrier()

    def body(k, carry):
        base = e0 + k * CB
        pltpu.sync_copy(ei_h.at[c, pl.ds(base, CB)], fidx)
        pltpu.sync_copy(ei_h.at[1 - c, pl.ds(base, CB)], tidx)
        pltpu.async_copy(ptbl_h.at[fidx], rows, sem).wait()
        pltpu.sync_copy(rows, acc.at[tidx], add=True)
        return carry

    lax.fori_loop(0, CHUNKS, body, 0)
    plsc.subcore_barrier()
    pltpu.sync_copy(acc.at[pl.ds(r0, ROWS_PER_TILE)],
                    agg2_o.at[c, pl.ds(r0, ROWS_PER_TILE)])


def _run_k3(ptbl, ei, zrows):
    k3 = functools.partial(
        pl.kernel,
        out_type=jax.ShapeDtypeStruct((NC, NP, HALF), F32),
        mesh=_sc_mesh(),
        scratch_types=[
            pltpu.VMEM((CB,), jnp.int32),
            pltpu.VMEM((CB,), jnp.int32),
            pltpu.VMEM((CB, HALF), F32),
            pltpu.VMEM_SHARED((NP, HALF), F32),
            pltpu.SemaphoreType.DMA,
        ],
    )(_k3_body)
    return k3(ptbl, ei, zrows)


def _k2_body(x_r, aa_r, ab_r, cnt_r,
             ws1, bs1, wsd1, bsd1, wds1, bds1,
             ws2, wpair,
             pself_r, pdir_r):
    cd = jnp.maximum(cnt_r[0, :, 0:1], 1.0)
    cs = jnp.maximum(cnt_r[1, :, 0:1], 1.0)
    a_s2d = jnp.concatenate([aa_r[0], ab_r[0]], axis=1) / cd
    a_d2s = jnp.concatenate([aa_r[1], ab_r[1]], axis=1) / cs
    xv = x_r[...]
    h = (jnp.dot(xv, ws1[...], preferred_element_type=F32) + bs1[...]
         + (1.0 - ALPHA_MIX) * (jnp.dot(a_s2d, wsd1[...], preferred_element_type=F32) + bsd1[...])
         + ALPHA_MIX * (jnp.dot(a_d2s, wds1[...], preferred_element_type=F32) + bds1[...]))
    h = SELU_SCALE * jnp.where(h > 0, h, SELU_ALPHA * (jnp.exp(h) - 1.0))
    pself_r[...] = jnp.dot(h, ws2[...], preferred_element_type=F32)
    pdir_r[...] = jnp.dot(h, wpair[...], preferred_element_type=F32)


def _run_k2(xp, agg_a, agg_b, cnt, ws1, bs1, wsd1, bsd1, wds1, bds1, ws2, wpair):
    R = 1024
    grid = (NP // R,)
    wspec = pl.BlockSpec((DIM_IN, DIM_HID), lambda i: (0, 0))
    bspec = pl.BlockSpec((1, DIM_HID), lambda i: (0, 0))
    return pl.pallas_call(
        _k2_body,
        grid=grid,
        in_specs=[
            pl.BlockSpec((R, DIM_IN), lambda i: (i, 0)),
            pl.BlockSpec((NC, R, HALF), lambda i: (0, i, 0)),
            pl.BlockSpec((NC, R, HALF), lambda i: (0, i, 0)),
            pl.BlockSpec((NC, R, HALF), lambda i: (0, i, 0)),
            wspec, bspec, wspec, bspec, wspec, bspec,
            pl.BlockSpec((DIM_HID, DIM_OUT), lambda i: (0, 0)),
            pl.BlockSpec((DIM_HID, 2 * DIM_OUT), lambda i: (0, 0)),
        ],
        out_specs=[
            pl.BlockSpec((R, DIM_OUT), lambda i: (i, 0)),
            pl.BlockSpec((R, 2 * DIM_OUT), lambda i: (i, 0)),
        ],
        out_shape=[
            jax.ShapeDtypeStruct((NP, DIM_OUT), F32),
            jax.ShapeDtypeStruct((NP, 2 * DIM_OUT), F32),
        ],
    )(xp, agg_a, agg_b, cnt, ws1, bs1, wsd1, bsd1, wds1, bds1, ws2, wpair)


def _k4_body(pself_r, agg2_r, cnt_r, bs2, bsd2, bds2, out_r):
    cd = jnp.maximum(cnt_r[0, :, 0:1], 1.0)
    cs = jnp.maximum(cnt_r[1, :, 0:1], 1.0)
    out_r[...] = (pself_r[...] + bs2[...]
                  + (1.0 - ALPHA_MIX) * (agg2_r[0, :, :DIM_OUT] / cd + bsd2[...])
                  + ALPHA_MIX * (agg2_r[1, :, DIM_OUT:] / cs + bds2[...]))


def _run_k4(pself, agg2, cnt, bs2, bsd2, bds2):
    R = 2000
    grid = (N_NODES // R,)
    bspec = pl.BlockSpec((1, DIM_OUT), lambda i: (0, 0))
    return pl.pallas_call(
        _k4_body,
        grid=grid,
        in_specs=[
            pl.BlockSpec((R, DIM_OUT), lambda i: (i, 0)),
            pl.BlockSpec((NC, R, HALF), lambda i: (0, i, 0)),
            pl.BlockSpec((NC, R, HALF), lambda i: (0, i, 0)),
            bspec, bspec, bspec,
        ],
        out_specs=pl.BlockSpec((R, DIM_OUT), lambda i: (i, 0)),
        out_shape=jax.ShapeDtypeStruct((N_NODES, DIM_OUT), F32),
    )(pself, agg2, cnt, bs2, bsd2, bds2)


def kernel(x, edge_index, W_self1, b_self1, W_s2d1, b_s2d1, W_d2s1, b_d2s1,
           W_self2, b_self2, W_s2d2, b_s2d2, W_d2s2, b_d2s2):
    xp = jnp.zeros((NP, DIM_IN), F32).at[:N_NODES].set(x)
    xa = xp[:, :HALF]
    xb = xp[:, HALF:]
    pad = jnp.full((2, EP - N_EDGES), NP - 1, jnp.int32)
    ei = jnp.concatenate([edge_index.astype(jnp.int32), pad], axis=1)
    zrows = jnp.zeros((ROWS_PER_TILE, HALF), F32)
    ones128 = jnp.ones((CB, HALF), F32)
    wpair = jnp.concatenate([W_s2d2, W_d2s2], axis=1)

    agg_a, agg_b, cnt = _run_k1(xa, xb, ei, zrows, ones128)

    pself, pdir = _run_k2(
        xp, agg_a, agg_b, cnt,
        W_self1, b_self1.reshape(1, DIM_HID),
        W_s2d1, b_s2d1.reshape(1, DIM_HID),
        W_d2s1, b_d2s1.reshape(1, DIM_HID),
        W_self2, wpair,
    )

    agg2 = _run_k3(pdir, ei, zrows)

    out = _run_k4(
        pself, agg2, cnt,
        b_self2.reshape(1, DIM_OUT),
        b_s2d2.reshape(1, DIM_OUT),
        b_d2s2.reshape(1, DIM_OUT),
    )
    return out


# speedup vs baseline: 4.1037x; 1.2185x over previous
import functools

import jax
import jax.numpy as jnp
from jax import lax
from jax.experimental import pallas as pl
from jax.experimental.pallas import tpu as pltpu
from jax.experimental.pallas import tpu_sc as plsc

F32 = jnp.float32

N_NODES = 10000
DIM_IN = 256
DIM_HID = 512
DIM_OUT = 64
N_EDGES = 160000
ALPHA_MIX = 0.5

NC = 2
NS = 16

NP = 10240
EP = 163840
CB = 128
EDGES_PER_TILE = EP // NS
CHUNKS = EDGES_PER_TILE // CB
HALF = DIM_IN // 2
ACCR = 10112
APT = ACCR // NS
PAD_IDX = ACCR - 1
SEGS = 2
SEGCH = CHUNKS // SEGS
CNT_GRP = 8

SELU_SCALE = 1.0507009873554805
SELU_ALPHA = 1.6732632423543772


def _sc_mesh():
    return plsc.VectorSubcoreMesh(
        core_axis_name="c", subcore_axis_name="s", num_cores=NC, num_subcores=NS
    )


def _zero_acc(zr_h, acc, r0):
    pltpu.sync_copy(zr_h, acc.at[pl.ds(r0, APT)])
    plsc.subcore_barrier()


def _writeout(acc, out_h, c, r0):
    plsc.subcore_barrier()
    pltpu.sync_copy(acc.at[pl.ds(r0, APT)], out_h.at[c, pl.ds(r0, APT)])


def _gather_scatter_pass(tbl_h, ei_h, c, s, fseg, tidx_all, rows, acc,
                         semA, semB):
    for seg in range(SEGS):
        pltpu.sync_copy(ei_h.at[c, s, pl.ds(seg * SEGCH, SEGCH)], fseg)
        pltpu.async_copy(tbl_h.at[fseg.at[0]], rows.at[0], semA)

        def grp(g, carry):
            k0 = 2 * g
            b1 = pltpu.async_copy(tbl_h.at[fseg.at[k0 + 1]], rows.at[1], semB)
            pltpu.make_async_copy(tbl_h.at[fseg.at[k0]], rows.at[0],
                                  semA).wait()
            pltpu.sync_copy(rows.at[0], acc.at[tidx_all.at[seg * SEGCH + k0]],
                            add=True)

            @pl.when(g < SEGCH // 2 - 1)
            def _():
                pltpu.async_copy(tbl_h.at[fseg.at[k0 + 2]], rows.at[0], semA)

            b1.wait()
            pltpu.sync_copy(rows.at[1],
                            acc.at[tidx_all.at[seg * SEGCH + k0 + 1]],
                            add=True)
            return carry

        lax.fori_loop(0, SEGCH // 2, grp, 0)


def _k1_body(xa_h, xb_h, ei_h, zr_h, on_h,
             agg_a, agg_b, cnt_o,
             fseg, tidx_all, rows, acc, semA, semB):
    c = lax.axis_index("c")
    s = lax.axis_index("s")
    r0 = s * APT

    pltpu.sync_copy(on_h, rows.at[0])
    pltpu.sync_copy(ei_h.at[1 - c, s], tidx_all)

    _zero_acc(zr_h, acc, r0)

    def cgrp(g, carry):
        k0 = g * CNT_GRP
        cps = [pltpu.async_copy(rows.at[0], acc.at[tidx_all.at[k0 + j]], semA,
                                add=True)
               for j in range(CNT_GRP)]
        for cp in cps:
            cp.wait()
        return carry

    lax.fori_loop(0, CHUNKS // CNT_GRP, cgrp, 0)
    _writeout(acc, cnt_o, c, r0)

    for tbl_h, out_h in ((xa_h, agg_a), (xb_h, agg_b)):
        _zero_acc(zr_h, acc, r0)
        _gather_scatter_pass(tbl_h, ei_h, c, s, fseg, tidx_all, rows, acc,
                             semA, semB)
        _writeout(acc, out_h, c, r0)


def _run_k1(xa, xb, ei, zrows, ones128):
    k1 = functools.partial(
        pl.kernel,
        out_type=[
            jax.ShapeDtypeStruct((NC, NP, HALF), F32),
            jax.ShapeDtypeStruct((NC, NP, HALF), F32),
            jax.ShapeDtypeStruct((NC, NP, HALF), F32),
        ],
        mesh=_sc_mesh(),
        scratch_types=[
            pltpu.VMEM((SEGCH, CB), jnp.int32),
            pltpu.VMEM((CHUNKS, CB), jnp.int32),
            pltpu.VMEM((2, CB, HALF), F32),
            pltpu.VMEM_SHARED((ACCR, HALF), F32),
            pltpu.SemaphoreType.DMA,
            pltpu.SemaphoreType.DMA,
        ],
    )(_k1_body)
    return k1(xa, xb, ei, zrows, ones128)


def _k3_body(ptbl_h, ei_h, zr_h, agg2_o, fseg, tidx_all, rows, acc,
             semA, semB):
    c = lax.axis_index("c")
    s = lax.axis_index("s")
    r0 = s * APT

    pltpu.sync_copy(ei_h.at[1 - c, s], tidx_all)
    _zero_acc(zr_h, acc, r0)
    _gather_scatter_pass(ptbl_h, ei_h, c, s, fseg, tidx_all, rows, acc,
                         semA, semB)
    _writeout(acc, agg2_o, c, r0)


def _run_k3(ptbl, ei, zrows):
    k3 = functools.partial(
        pl.kernel,
        out_type=jax.ShapeDtypeStruct((NC, NP, HALF), F32),
        mesh=_sc_mesh(),
        scratch_types=[
            pltpu.VMEM((SEGCH, CB), jnp.int32),
            pltpu.VMEM((CHUNKS, CB), jnp.int32),
            pltpu.VMEM((2, CB, HALF), F32),
            pltpu.VMEM_SHARED((ACCR, HALF), F32),
            pltpu.SemaphoreType.DMA,
            pltpu.SemaphoreType.DMA,
        ],
    )(_k3_body)
    return k3(ptbl, ei, zrows)


def _k2_body(x_r, aa_r, ab_r, cnt_r,
             ws1, bs1, wsd1, bsd1, wds1, bds1,
             ws2, wpair,
             pself_r, pdir_r):
    cd = jnp.maximum(cnt_r[0, :, 0:1], 1.0)
    cs = jnp.maximum(cnt_r[1, :, 0:1], 1.0)
    a_s2d = jnp.concatenate([aa_r[0], ab_r[0]], axis=1) / cd
    a_d2s = jnp.concatenate([aa_r[1], ab_r[1]], axis=1) / cs
    xv = x_r[...]
    h = (jnp.dot(xv, ws1[...], preferred_element_type=F32) + bs1[...]
         + (1.0 - ALPHA_MIX) * (jnp.dot(a_s2d, wsd1[...], preferred_element_type=F32) + bsd1[...])
         + ALPHA_MIX * (jnp.dot(a_d2s, wds1[...], preferred_element_type=F32) + bds1[...]))
    h = SELU_SCALE * jnp.where(h > 0, h, SELU_ALPHA * (jnp.exp(h) - 1.0))
    pself_r[...] = jnp.dot(h, ws2[...], preferred_element_type=F32)
    pdir_r[...] = jnp.dot(h, wpair[...], preferred_element_type=F32)


def _run_k2(xp, agg_a, agg_b, cnt, ws1, bs1, wsd1, bsd1, wds1, bds1, ws2, wpair):
    R = 1024
    grid = (NP // R,)
    wspec = pl.BlockSpec((DIM_IN, DIM_HID), lambda i: (0, 0))
    bspec = pl.BlockSpec((1, DIM_HID), lambda i: (0, 0))
    return pl.pallas_call(
        _k2_body,
        grid=grid,
        in_specs=[
            pl.BlockSpec((R, DIM_IN), lambda i: (i, 0)),
            pl.BlockSpec((NC, R, HALF), lambda i: (0, i, 0)),
            pl.BlockSpec((NC, R, HALF), lambda i: (0, i, 0)),
            pl.BlockSpec((NC, R, HALF), lambda i: (0, i, 0)),
            wspec, bspec, wspec, bspec, wspec, bspec,
            pl.BlockSpec((DIM_HID, DIM_OUT), lambda i: (0, 0)),
            pl.BlockSpec((DIM_HID, 2 * DIM_OUT), lambda i: (0, 0)),
        ],
        out_specs=[
            pl.BlockSpec((R, DIM_OUT), lambda i: (i, 0)),
            pl.BlockSpec((R, 2 * DIM_OUT), lambda i: (i, 0)),
        ],
        out_shape=[
            jax.ShapeDtypeStruct((NP, DIM_OUT), F32),
            jax.ShapeDtypeStruct((NP, 2 * DIM_OUT), F32),
        ],
    )(xp, agg_a, agg_b, cnt, ws1, bs1, wsd1, bsd1, wds1, bds1, ws2, wpair)


def _k4_body(pself_r, agg2_r, cnt_r, bs2, bsd2, bds2, out_r):
    cd = jnp.maximum(cnt_r[0, :, 0:1], 1.0)
    cs = jnp.maximum(cnt_r[1, :, 0:1], 1.0)
    out_r[...] = (pself_r[...] + bs2[...]
                  + (1.0 - ALPHA_MIX) * (agg2_r[0, :, :DIM_OUT] / cd + bsd2[...])
                  + ALPHA_MIX * (agg2_r[1, :, DIM_OUT:] / cs + bds2[...]))


def _run_k4(pself, agg2, cnt, bs2, bsd2, bds2):
    R = 2000
    grid = (N_NODES // R,)
    bspec = pl.BlockSpec((1, DIM_OUT), lambda i: (0, 0))
    return pl.pallas_call(
        _k4_body,
        grid=grid,
        in_specs=[
            pl.BlockSpec((R, DIM_OUT), lambda i: (i, 0)),
            pl.BlockSpec((NC, R, HALF), lambda i: (0, i, 0)),
            pl.BlockSpec((NC, R, HALF), lambda i: (0, i, 0)),
            bspec, bspec, bspec,
        ],
        out_specs=pl.BlockSpec((R, DIM_OUT), lambda i: (i, 0)),
        out_shape=jax.ShapeDtypeStruct((N_NODES, DIM_OUT), F32),
    )(pself, agg2, cnt, bs2, bsd2, bds2)


def kernel(x, edge_index, W_self1, b_self1, W_s2d1, b_s2d1, W_d2s1, b_d2s1,
           W_self2, b_self2, W_s2d2, b_s2d2, W_d2s2, b_d2s2):
    xp = jnp.zeros((NP, DIM_IN), F32).at[:N_NODES].set(x)
    xa = xp[:, :HALF]
    xb = xp[:, HALF:]
    pad = jnp.full((2, EP - N_EDGES), PAD_IDX, jnp.int32)
    ei = jnp.concatenate([edge_index.astype(jnp.int32), pad], axis=1)
    ei = ei.reshape(2, NS, CHUNKS, CB)
    zrows = jnp.zeros((APT, HALF), F32)
    ones128 = jnp.ones((CB, HALF), F32)
    wpair = jnp.concatenate([W_s2d2, W_d2s2], axis=1)

    agg_a, agg_b, cnt = _run_k1(xa, xb, ei, zrows, ones128)

    pself, pdir = _run_k2(
        xp, agg_a, agg_b, cnt,
        W_self1, b_self1.reshape(1, DIM_HID),
        W_s2d1, b_s2d1.reshape(1, DIM_HID),
        W_d2s1, b_d2s1.reshape(1, DIM_HID),
        W_self2, wpair,
    )

    agg2 = _run_k3(pdir, ei, zrows)

    out = _run_k4(
        pself, agg2, cnt,
        b_self2.reshape(1, DIM_OUT),
        b_s2d2.reshape(1, DIM_OUT),
        b_d2s2.reshape(1, DIM_OUT),
    )
    return out


# speedup vs baseline: 4.5683x; 1.1132x over previous
import functools

import jax
import jax.numpy as jnp
from jax import lax
from jax.experimental import pallas as pl
from jax.experimental.pallas import tpu as pltpu
from jax.experimental.pallas import tpu_sc as plsc

F32 = jnp.float32

N_NODES = 10000
DIM_IN = 256
DIM_HID = 512
DIM_OUT = 64
N_EDGES = 160000
ALPHA_MIX = 0.5

NC = 2
NS = 16

NP = 10240
EP = 163840
CB = 128
EDGES_PER_TILE = EP // NS
CHUNKS = EDGES_PER_TILE // CB
HALF = DIM_IN // 2
ACCR = 10112
APT = ACCR // NS
PAD_IDX = ACCR - 1
SEGS = 2
SEGCH = CHUNKS // SEGS
CNT_GRP = 8

SELU_SCALE = 1.0507009873554805
SELU_ALPHA = 1.6732632423543772


def _sc_mesh():
    return plsc.VectorSubcoreMesh(
        core_axis_name="c", subcore_axis_name="s", num_cores=NC, num_subcores=NS
    )


def _zero_acc(zr_h, acc, r0):
    pltpu.sync_copy(zr_h, acc.at[pl.ds(r0, APT)])
    plsc.subcore_barrier()


def _writeout(acc, out_h, c, r0):
    plsc.subcore_barrier()
    pltpu.sync_copy(acc.at[pl.ds(r0, APT)], out_h.at[c, pl.ds(r0, APT)])


def _pipe_pass(tbl_h, fidx2d, tidx2d, nch, toff, rows, acc, semA, semB):
    pltpu.async_copy(tbl_h.at[fidx2d.at[0]], rows.at[0], semA)

    def grp(g, carry):
        k0 = 2 * g
        b1 = pltpu.async_copy(tbl_h.at[fidx2d.at[k0 + 1]], rows.at[1], semB)
        pltpu.make_async_copy(tbl_h.at[fidx2d.at[k0]], rows.at[0],
                              semA).wait()
        pltpu.sync_copy(rows.at[0], acc.at[tidx2d.at[toff + k0]], add=True)

        @pl.when(g < nch // 2 - 1)
        def _():
            pltpu.async_copy(tbl_h.at[fidx2d.at[k0 + 2]], rows.at[0], semA)

        b1.wait()
        pltpu.sync_copy(rows.at[1], acc.at[tidx2d.at[toff + k0 + 1]],
                        add=True)
        return carry

    lax.fori_loop(0, nch // 2, grp, 0)


def _k1_body(xa_h, xb_h, ei_h, zr_h, on_h,
             agg_a, agg_b, cnt_o,
             fseg, tidx_all, rows, acc, semA, semB):
    c = lax.axis_index("c")
    s = lax.axis_index("s")
    r0 = s * APT

    pltpu.sync_copy(on_h, rows.at[0])
    pltpu.sync_copy(ei_h.at[1 - c, s], tidx_all)

    _zero_acc(zr_h, acc, r0)

    def cgrp(g, carry):
        k0 = g * CNT_GRP
        cps = [pltpu.async_copy(rows.at[0], acc.at[tidx_all.at[k0 + j]], semA,
                                add=True)
               for j in range(CNT_GRP)]
        for cp in cps:
            cp.wait()
        return carry

    lax.fori_loop(0, CHUNKS // CNT_GRP, cgrp, 0)
    _writeout(acc, cnt_o, c, r0)

    for tbl_h, out_h in ((xa_h, agg_a), (xb_h, agg_b)):
        _zero_acc(zr_h, acc, r0)
        for seg in range(SEGS):
            pltpu.sync_copy(ei_h.at[c, s, pl.ds(seg * SEGCH, SEGCH)], fseg)
            _pipe_pass(tbl_h, fseg, tidx_all, SEGCH, seg * SEGCH, rows, acc,
                       semA, semB)
        _writeout(acc, out_h, c, r0)


def _run_k1(xa, xb, ei, zrows, ones128):
    k1 = functools.partial(
        pl.kernel,
        out_type=[
            jax.ShapeDtypeStruct((NC, NP, HALF), F32),
            jax.ShapeDtypeStruct((NC, NP, HALF), F32),
            jax.ShapeDtypeStruct((NC, NP, HALF), F32),
        ],
        mesh=_sc_mesh(),
        scratch_types=[
            pltpu.VMEM((SEGCH, CB), jnp.int32),
            pltpu.VMEM((CHUNKS, CB), jnp.int32),
            pltpu.VMEM((2, CB, HALF), F32),
            pltpu.VMEM_SHARED((ACCR, HALF), F32),
            pltpu.SemaphoreType.DMA,
            pltpu.SemaphoreType.DMA,
        ],
    )(_k1_body)
    return k1(xa, xb, ei, zrows, ones128)


def _make_k3_body(frm):
    def body(ptbl_h, ei_h, zr_h, out_o, fseg, tseg, rows, acc, semA, semB):
        c = lax.axis_index("c")
        s = lax.axis_index("s")
        r0 = s * APT

        pltpu.sync_copy(ei_h.at[frm, s, pl.ds(c * SEGCH, SEGCH)], fseg)
        pltpu.sync_copy(ei_h.at[1 - frm, s, pl.ds(c * SEGCH, SEGCH)], tseg)
        _zero_acc(zr_h, acc, r0)
        _pipe_pass(ptbl_h, fseg, tseg, SEGCH, 0, rows, acc, semA, semB)
        _writeout(acc, out_o, c, r0)

    return body


def _run_k3(ptbl, ei, zrows64, frm):
    k3 = functools.partial(
        pl.kernel,
        out_type=jax.ShapeDtypeStruct((NC, NP, DIM_OUT), F32),
        mesh=_sc_mesh(),
        scratch_types=[
            pltpu.VMEM((SEGCH, CB), jnp.int32),
            pltpu.VMEM((SEGCH, CB), jnp.int32),
            pltpu.VMEM((2, CB, DIM_OUT), F32),
            pltpu.VMEM_SHARED((ACCR, DIM_OUT), F32),
            pltpu.SemaphoreType.DMA,
            pltpu.SemaphoreType.DMA,
        ],
        compiler_params=pltpu.CompilerParams(use_tc_tiling_on_sc=False),
    )(_make_k3_body(frm))
    return k3(ptbl, ei, zrows64)


def _k2_body(x_r, aa_r, ab_r, cnt_r,
             ws1, bs1, wsd1, bsd1, wds1, bds1,
             ws2, wsd2, wds2,
             pself_r, p0_r, p1_r):
    cd = jnp.maximum(cnt_r[0, :, 0:1], 1.0)
    cs = jnp.maximum(cnt_r[1, :, 0:1], 1.0)
    a_s2d = jnp.concatenate([aa_r[0], ab_r[0]], axis=1) / cd
    a_d2s = jnp.concatenate([aa_r[1], ab_r[1]], axis=1) / cs
    xv = x_r[...]
    h = (jnp.dot(xv, ws1[...], preferred_element_type=F32) + bs1[...]
         + (1.0 - ALPHA_MIX) * (jnp.dot(a_s2d, wsd1[...], preferred_element_type=F32) + bsd1[...])
         + ALPHA_MIX * (jnp.dot(a_d2s, wds1[...], preferred_element_type=F32) + bds1[...]))
    h = SELU_SCALE * jnp.where(h > 0, h, SELU_ALPHA * (jnp.exp(h) - 1.0))
    pself_r[...] = jnp.dot(h, ws2[...], preferred_element_type=F32)
    p0_r[...] = jnp.dot(h, wsd2[...], preferred_element_type=F32)
    p1_r[...] = jnp.dot(h, wds2[...], preferred_element_type=F32)


def _run_k2(xp, agg_a, agg_b, cnt, ws1, bs1, wsd1, bsd1, wds1, bds1, ws2, wsd2, wds2):
    R = 1024
    grid = (NP // R,)
    wspec = pl.BlockSpec((DIM_IN, DIM_HID), lambda i: (0, 0))
    bspec = pl.BlockSpec((1, DIM_HID), lambda i: (0, 0))
    return pl.pallas_call(
        _k2_body,
        grid=grid,
        in_specs=[
            pl.BlockSpec((R, DIM_IN), lambda i: (i, 0)),
            pl.BlockSpec((NC, R, HALF), lambda i: (0, i, 0)),
            pl.BlockSpec((NC, R, HALF), lambda i: (0, i, 0)),
            pl.BlockSpec((NC, R, HALF), lambda i: (0, i, 0)),
            wspec, bspec, wspec, bspec, wspec, bspec,
            pl.BlockSpec((DIM_HID, DIM_OUT), lambda i: (0, 0)),
            pl.BlockSpec((DIM_HID, DIM_OUT), lambda i: (0, 0)),
            pl.BlockSpec((DIM_HID, DIM_OUT), lambda i: (0, 0)),
        ],
        out_specs=[
            pl.BlockSpec((R, DIM_OUT), lambda i: (i, 0)),
            pl.BlockSpec((R, DIM_OUT), lambda i: (i, 0)),
            pl.BlockSpec((R, DIM_OUT), lambda i: (i, 0)),
        ],
        out_shape=[
            jax.ShapeDtypeStruct((NP, DIM_OUT), F32),
            jax.ShapeDtypeStruct((NP, DIM_OUT), F32),
            jax.ShapeDtypeStruct((NP, DIM_OUT), F32),
        ],
    )(xp, agg_a, agg_b, cnt, ws1, bs1, wsd1, bsd1, wds1, bds1, ws2, wsd2, wds2)


def _k4_body(pself_r, agg2a_r, agg2b_r, cnt_r, bs2, bsd2, bds2, out_r):
    cd = jnp.maximum(cnt_r[0, :, 0:1], 1.0)
    cs = jnp.maximum(cnt_r[1, :, 0:1], 1.0)
    s2d = agg2a_r[0] + agg2a_r[1]
    d2s = agg2b_r[0] + agg2b_r[1]
    out_r[...] = (pself_r[...] + bs2[...]
                  + (1.0 - ALPHA_MIX) * (s2d / cd + bsd2[...])
                  + ALPHA_MIX * (d2s / cs + bds2[...]))


def _run_k4(pself, agg2a, agg2b, cnt, bs2, bsd2, bds2):
    R = 2000
    grid = (N_NODES // R,)
    bspec = pl.BlockSpec((1, DIM_OUT), lambda i: (0, 0))
    return pl.pallas_call(
        _k4_body,
        grid=grid,
        in_specs=[
            pl.BlockSpec((R, DIM_OUT), lambda i: (i, 0)),
            pl.BlockSpec((NC, R, DIM_OUT), lambda i: (0, i, 0)),
            pl.BlockSpec((NC, R, DIM_OUT), lambda i: (0, i, 0)),
            pl.BlockSpec((NC, R, HALF), lambda i: (0, i, 0)),
            bspec, bspec, bspec,
        ],
        out_specs=pl.BlockSpec((R, DIM_OUT), lambda i: (i, 0)),
        out_shape=jax.ShapeDtypeStruct((N_NODES, DIM_OUT), F32),
    )(pself, agg2a, agg2b, cnt, bs2, bsd2, bds2)


def kernel(x, edge_index, W_self1, b_self1, W_s2d1, b_s2d1, W_d2s1, b_d2s1,
           W_self2, b_self2, W_s2d2, b_s2d2, W_d2s2, b_d2s2):
    xp = jnp.zeros((NP, DIM_IN), F32).at[:N_NODES].set(x)
    xa = xp[:, :HALF]
    xb = xp[:, HALF:]
    pad = jnp.full((2, EP - N_EDGES), PAD_IDX, jnp.int32)
    ei = jnp.concatenate([edge_index.astype(jnp.int32), pad], axis=1)
    ei = ei.reshape(2, NS, CHUNKS, CB)
    zrows = jnp.zeros((APT, HALF), F32)
    zrows64 = jnp.zeros((APT, DIM_OUT), F32)
    ones128 = jnp.ones((CB, HALF), F32)

    agg_a, agg_b, cnt = _run_k1(xa, xb, ei, zrows, ones128)

    pself, pdir0, pdir1 = _run_k2(
        xp, agg_a, agg_b, cnt,
        W_self1, b_self1.reshape(1, DIM_HID),
        W_s2d1, b_s2d1.reshape(1, DIM_HID),
        W_d2s1, b_d2s1.reshape(1, DIM_HID),
        W_self2, W_s2d2, W_d2s2,
    )

    agg2a = _run_k3(pdir0, ei, zrows64, 0)
    agg2b = _run_k3(pdir1, ei, zrows64, 1)

    out = _run_k4(
        pself, agg2a, agg2b, cnt,
        b_self2.reshape(1, DIM_OUT),
        b_s2d2.reshape(1, DIM_OUT),
        b_d2s2.reshape(1, DIM_OUT),
    )
    return out


# speedup vs baseline: 4.5829x; 1.0032x over previous
import functools

import jax
import jax.numpy as jnp
from jax import lax
from jax.experimental import pallas as pl
from jax.experimental.pallas import tpu as pltpu
from jax.experimental.pallas import tpu_sc as plsc

F32 = jnp.float32

N_NODES = 10000
DIM_IN = 256
DIM_HID = 512
DIM_OUT = 64
N_EDGES = 160000
ALPHA_MIX = 0.5

NC = 2
NS = 16

NP = 10240
EP = 163840
CB = 64
EDGES_PER_TILE = EP // NS
CHUNKS = EDGES_PER_TILE // CB
HALF = DIM_IN // 2
ACCR = 10112
APT = ACCR // NS
PAD_IDX = ACCR - 1
SEGS = 2
SEGCH = CHUNKS // SEGS
CNT_GRP = 8

SELU_SCALE = 1.0507009873554805
SELU_ALPHA = 1.6732632423543772

_UNTILED = pltpu.CompilerParams(use_tc_tiling_on_sc=False)


def _sc_mesh():
    return plsc.VectorSubcoreMesh(
        core_axis_name="c", subcore_axis_name="s", num_cores=NC, num_subcores=NS
    )


def _zero_acc(zr_h, acc, r0):
    pltpu.sync_copy(zr_h, acc.at[pl.ds(r0, APT)])
    plsc.subcore_barrier()


def _writeout(acc, out_h, c, r0):
    plsc.subcore_barrier()
    pltpu.sync_copy(acc.at[pl.ds(r0, APT)], out_h.at[c, pl.ds(r0, APT)])


def _pipe_pass(tbl_h, fidx2d, tidx2d, nch, toff, rows, acc, sems):
    s0, s1, s2, s3 = sems
    pltpu.async_copy(tbl_h.at[fidx2d.at[0]], rows.at[0], s0)
    pltpu.async_copy(tbl_h.at[fidx2d.at[1]], rows.at[1], s1)
    pltpu.async_copy(tbl_h.at[fidx2d.at[2]], rows.at[2], s2)
    ng = nch // 4

    def grp(g, carry):
        k0 = 4 * g
        for j in range(4):
            nb = (j + 3) % 4
            if j == 0:
                pltpu.async_copy(tbl_h.at[fidx2d.at[k0 + 3]], rows.at[3], s3)
            else:
                @pl.when(g < ng - 1)
                def _(nk=k0 + j + 3, nb=nb, sn=sems[nb]):
                    pltpu.async_copy(tbl_h.at[fidx2d.at[nk]], rows.at[nb], sn)
            pltpu.make_async_copy(tbl_h.at[fidx2d.at[k0 + j]], rows.at[j],
                                  sems[j]).wait()
            pltpu.sync_copy(rows.at[j], acc.at[tidx2d.at[toff + k0 + j]],
                            add=True)
        return carry

    lax.fori_loop(0, ng, grp, 0)


def _k1_body(xa_h, xb_h, ei_h, zr_h, on_h,
             agg_a, agg_b, cnt_o,
             fseg, tidx_all, rows, acc, s0, s1, s2, s3):
    c = lax.axis_index("c")
    s = lax.axis_index("s")
    r0 = s * APT
    sems = (s0, s1, s2, s3)

    pltpu.sync_copy(on_h, rows.at[0])
    pltpu.sync_copy(ei_h.at[1 - c, s], tidx_all)

    _zero_acc(zr_h, acc, r0)

    def cgrp(g, carry):
        k0 = g * CNT_GRP
        cps = [pltpu.async_copy(rows.at[0], acc.at[tidx_all.at[k0 + j]], s0,
                                add=True)
               for j in range(CNT_GRP)]
        for cp in cps:
            cp.wait()
        return carry

    lax.fori_loop(0, CHUNKS // CNT_GRP, cgrp, 0)
    _writeout(acc, cnt_o, c, r0)

    for tbl_h, out_h in ((xa_h, agg_a), (xb_h, agg_b)):
        _zero_acc(zr_h, acc, r0)
        for seg in range(SEGS):
            pltpu.sync_copy(ei_h.at[c, s, pl.ds(seg * SEGCH, SEGCH)], fseg)
            _pipe_pass(tbl_h, fseg, tidx_all, SEGCH, seg * SEGCH, rows, acc,
                       sems)
        _writeout(acc, out_h, c, r0)


def _run_k1(xa, xb, ei, zrows, ones_rows):
    k1 = functools.partial(
        pl.kernel,
        out_type=[
            jax.ShapeDtypeStruct((NC, NP, HALF), F32),
            jax.ShapeDtypeStruct((NC, NP, HALF), F32),
            jax.ShapeDtypeStruct((NC, NP, HALF), F32),
        ],
        mesh=_sc_mesh(),
        scratch_types=[
            pltpu.VMEM((SEGCH, CB), jnp.int32),
            pltpu.VMEM((CHUNKS, CB), jnp.int32),
            pltpu.VMEM((4, CB, HALF), F32),
            pltpu.VMEM_SHARED((ACCR, HALF), F32),
            pltpu.SemaphoreType.DMA,
            pltpu.SemaphoreType.DMA,
            pltpu.SemaphoreType.DMA,
            pltpu.SemaphoreType.DMA,
        ],
        compiler_params=_UNTILED,
    )(_k1_body)
    return k1(xa, xb, ei, zrows, ones_rows)


def _make_k3_body(frm):
    def body(ptbl_h, ei_h, zr_h, out_o, fseg, tseg, rows, acc,
             s0, s1, s2, s3):
        c = lax.axis_index("c")
        s = lax.axis_index("s")
        r0 = s * APT

        pltpu.sync_copy(ei_h.at[frm, s, pl.ds(c * SEGCH, SEGCH)], fseg)
        pltpu.sync_copy(ei_h.at[1 - frm, s, pl.ds(c * SEGCH, SEGCH)], tseg)
        _zero_acc(zr_h, acc, r0)
        _pipe_pass(ptbl_h, fseg, tseg, SEGCH, 0, rows, acc, (s0, s1, s2, s3))
        _writeout(acc, out_o, c, r0)

    return body


def _run_k3(ptbl, ei, zrows64, frm):
    k3 = functools.partial(
        pl.kernel,
        out_type=jax.ShapeDtypeStruct((NC, NP, DIM_OUT), F32),
        mesh=_sc_mesh(),
        scratch_types=[
            pltpu.VMEM((SEGCH, CB), jnp.int32),
            pltpu.VMEM((SEGCH, CB), jnp.int32),
            pltpu.VMEM((4, CB, DIM_OUT), F32),
            pltpu.VMEM_SHARED((ACCR, DIM_OUT), F32),
            pltpu.SemaphoreType.DMA,
            pltpu.SemaphoreType.DMA,
            pltpu.SemaphoreType.DMA,
            pltpu.SemaphoreType.DMA,
        ],
        compiler_params=_UNTILED,
    )(_make_k3_body(frm))
    return k3(ptbl, ei, zrows64)


def _k2_body(x_r, aa_r, ab_r, cnt_r,
             ws1, bs1, wsd1, bsd1, wds1, bds1,
             ws2, wsd2, wds2,
             pself_r, p0_r, p1_r):
    cd = jnp.maximum(cnt_r[0, :, 0:1], 1.0)
    cs = jnp.maximum(cnt_r[1, :, 0:1], 1.0)
    a_s2d = jnp.concatenate([aa_r[0], ab_r[0]], axis=1) / cd
    a_d2s = jnp.concatenate([aa_r[1], ab_r[1]], axis=1) / cs
    xv = x_r[...]
    h = (jnp.dot(xv, ws1[...], preferred_element_type=F32) + bs1[...]
         + (1.0 - ALPHA_MIX) * (jnp.dot(a_s2d, wsd1[...], preferred_element_type=F32) + bsd1[...])
         + ALPHA_MIX * (jnp.dot(a_d2s, wds1[...], preferred_element_type=F32) + bds1[...]))
    h = SELU_SCALE * jnp.where(h > 0, h, SELU_ALPHA * (jnp.exp(h) - 1.0))
    pself_r[...] = jnp.dot(h, ws2[...], preferred_element_type=F32)
    p0_r[...] = jnp.dot(h, wsd2[...], preferred_element_type=F32)
    p1_r[...] = jnp.dot(h, wds2[...], preferred_element_type=F32)


def _run_k2(xp, agg_a, agg_b, cnt, ws1, bs1, wsd1, bsd1, wds1, bds1, ws2, wsd2, wds2):
    R = 1024
    grid = (NP // R,)
    wspec = pl.BlockSpec((DIM_IN, DIM_HID), lambda i: (0, 0))
    bspec = pl.BlockSpec((1, DIM_HID), lambda i: (0, 0))
    w2spec = pl.BlockSpec((DIM_HID, DIM_OUT), lambda i: (0, 0))
    return pl.pallas_call(
        _k2_body,
        grid=grid,
        in_specs=[
            pl.BlockSpec((R, DIM_IN), lambda i: (i, 0)),
            pl.BlockSpec((NC, R, HALF), lambda i: (0, i, 0)),
            pl.BlockSpec((NC, R, HALF), lambda i: (0, i, 0)),
            pl.BlockSpec((NC, R, HALF), lambda i: (0, i, 0)),
            wspec, bspec, wspec, bspec, wspec, bspec,
            w2spec, w2spec, w2spec,
        ],
        out_specs=[
            pl.BlockSpec((R, DIM_OUT), lambda i: (i, 0)),
            pl.BlockSpec((R, DIM_OUT), lambda i: (i, 0)),
            pl.BlockSpec((R, DIM_OUT), lambda i: (i, 0)),
        ],
        out_shape=[
            jax.ShapeDtypeStruct((NP, DIM_OUT), F32),
            jax.ShapeDtypeStruct((NP, DIM_OUT), F32),
            jax.ShapeDtypeStruct((NP, DIM_OUT), F32),
        ],
    )(xp, agg_a, agg_b, cnt, ws1, bs1, wsd1, bsd1, wds1, bds1, ws2, wsd2, wds2)


def _k4_body(pself_r, agg2a_r, agg2b_r, cnt_r, bs2, bsd2, bds2, out_r):
    cd = jnp.maximum(cnt_r[0, :, 0:1], 1.0)
    cs = jnp.maximum(cnt_r[1, :, 0:1], 1.0)
    s2d = agg2a_r[0] + agg2a_r[1]
    d2s = agg2b_r[0] + agg2b_r[1]
    out_r[...] = (pself_r[...] + bs2[...]
                  + (1.0 - ALPHA_MIX) * (s2d / cd + bsd2[...])
                  + ALPHA_MIX * (d2s / cs + bds2[...]))


def _run_k4(pself, agg2a, agg2b, cnt, bs2, bsd2, bds2):
    R = 2000
    grid = (N_NODES // R,)
    bspec = pl.BlockSpec((1, DIM_OUT), lambda i: (0, 0))
    return pl.pallas_call(
        _k4_body,
        grid=grid,
        in_specs=[
            pl.BlockSpec((R, DIM_OUT), lambda i: (i, 0)),
            pl.BlockSpec((NC, R, DIM_OUT), lambda i: (0, i, 0)),
            pl.BlockSpec((NC, R, DIM_OUT), lambda i: (0, i, 0)),
            pl.BlockSpec((NC, R, HALF), lambda i: (0, i, 0)),
            bspec, bspec, bspec,
        ],
        out_specs=pl.BlockSpec((R, DIM_OUT), lambda i: (i, 0)),
        out_shape=jax.ShapeDtypeStruct((N_NODES, DIM_OUT), F32),
    )(pself, agg2a, agg2b, cnt, bs2, bsd2, bds2)


def kernel(x, edge_index, W_self1, b_self1, W_s2d1, b_s2d1, W_d2s1, b_d2s1,
           W_self2, b_self2, W_s2d2, b_s2d2, W_d2s2, b_d2s2):
    xp = jnp.zeros((NP, DIM_IN), F32).at[:N_NODES].set(x)
    xa = xp[:, :HALF]
    xb = xp[:, HALF:]
    pad = jnp.full((2, EP - N_EDGES), PAD_IDX, jnp.int32)
    ei = jnp.concatenate([edge_index.astype(jnp.int32), pad], axis=1)
    ei = ei.reshape(2, NS, CHUNKS, CB)
    zrows = jnp.zeros((APT, HALF), F32)
    zrows64 = jnp.zeros((APT, DIM_OUT), F32)
    ones_rows = jnp.ones((CB, HALF), F32)

    agg_a, agg_b, cnt = _run_k1(xa, xb, ei, zrows, ones_rows)

    pself, pdir0, pdir1 = _run_k2(
        xp, agg_a, agg_b, cnt,
        W_self1, b_self1.reshape(1, DIM_HID),
        W_s2d1, b_s2d1.reshape(1, DIM_HID),
        W_d2s1, b_d2s1.reshape(1, DIM_HID),
        W_self2, W_s2d2, W_d2s2,
    )

    agg2a = _run_k3(pdir0, ei, zrows64, 0)
    agg2b = _run_k3(pdir1, ei, zrows64, 1)

    out = _run_k4(
        pself, agg2a, agg2b, cnt,
        b_self2.reshape(1, DIM_OUT),
        b_s2d2.reshape(1, DIM_OUT),
        b_d2s2.reshape(1, DIM_OUT),
    )
    return out


# speedup vs baseline: 4.6218x; 1.0085x over previous
import functools

import jax
import jax.numpy as jnp
from jax import lax
from jax.experimental import pallas as pl
from jax.experimental.pallas import tpu as pltpu
from jax.experimental.pallas import tpu_sc as plsc

F32 = jnp.float32

N_NODES = 10000
DIM_IN = 256
DIM_HID = 512
DIM_OUT = 64
N_EDGES = 160000
ALPHA_MIX = 0.5

NC = 2
NS = 16

NP = 10240
EP = 163840
CB = 64
EDGES_PER_TILE = EP // NS
CHUNKS = EDGES_PER_TILE // CB
HALF = DIM_IN // 2
ACCR = 10112
APT = ACCR // NS
PAD_IDX = ACCR - 1
SEGS = 2
SEGCH = CHUNKS // SEGS
CNT_GRP = 8

SELU_SCALE = 1.0507009873554805
SELU_ALPHA = 1.6732632423543772

_UNTILED = pltpu.CompilerParams(use_tc_tiling_on_sc=False)
_UNTILED_NOLAYOUT = pltpu.CompilerParams(use_tc_tiling_on_sc=False,
                                         needs_layout_passes=False)

_PERM = tuple(
    128 * h + 32 * q + 2 * p + r
    for h in range(2) for q in range(4) for r in range(2) for p in range(16)
)


def _sc_mesh():
    return plsc.VectorSubcoreMesh(
        core_axis_name="c", subcore_axis_name="s", num_cores=NC, num_subcores=NS
    )


def _zero_acc(zr_h, acc, r0):
    pltpu.sync_copy(zr_h, acc.at[pl.ds(r0, APT)])
    plsc.subcore_barrier()


def _writeout(acc, out_h, c, r0):
    plsc.subcore_barrier()
    pltpu.sync_copy(acc.at[pl.ds(r0, APT)], out_h.at[c, pl.ds(r0, APT)])


def _unpack_rows(rows, j, fbuf):
    def crow(r, carry):
        for q in range(HALF // 32):
            v = rows[j, r, pl.ds(16 * q, 16)]
            lo = plsc.bitcast(lax.shift_left(v, 16), F32)
            hi = plsc.bitcast(lax.bitwise_and(v, jnp.int32(-65536)), F32)
            fbuf[r, pl.ds(32 * q, 16)] = lo
            fbuf[r, pl.ds(32 * q + 16, 16)] = hi
        return carry

    lax.fori_loop(0, CB, crow, 0)


def _pipe_pass(tbl_h, fidx2d, tidx2d, nch, toff, rows, acc, sems, fbuf=None):
    s0, s1, s2, s3 = sems
    pltpu.async_copy(tbl_h.at[fidx2d.at[0]], rows.at[0], s0)
    pltpu.async_copy(tbl_h.at[fidx2d.at[1]], rows.at[1], s1)
    pltpu.async_copy(tbl_h.at[fidx2d.at[2]], rows.at[2], s2)
    ng = nch // 4

    def grp(g, carry):
        k0 = 4 * g
        for j in range(4):
            nb = (j + 3) % 4
            if j == 0:
                pltpu.async_copy(tbl_h.at[fidx2d.at[k0 + 3]], rows.at[3], s3)
            else:
                @pl.when(g < ng - 1)
                def _(nk=k0 + j + 3, nb=nb, sn=sems[nb]):
                    pltpu.async_copy(tbl_h.at[fidx2d.at[nk]], rows.at[nb], sn)
            pltpu.make_async_copy(tbl_h.at[fidx2d.at[k0 + j]], rows.at[j],
                                  sems[j]).wait()
            if fbuf is None:
                pltpu.sync_copy(rows.at[j], acc.at[tidx2d.at[toff + k0 + j]],
                                add=True)
            else:
                _unpack_rows(rows, j, fbuf)
                pltpu.sync_copy(fbuf, acc.at[tidx2d.at[toff + k0 + j]],
                                add=True)
        return carry

    lax.fori_loop(0, ng, grp, 0)


def _k1_body(xa_h, xb_h, ei_h, zr_h, on_h,
             agg_a, agg_b, cnt_o,
             fseg, tidx_all, rows, fbuf, acc, s0, s1, s2, s3):
    c = lax.axis_index("c")
    s = lax.axis_index("s")
    r0 = s * APT
    sems = (s0, s1, s2, s3)

    pltpu.sync_copy(on_h, fbuf)
    pltpu.sync_copy(ei_h.at[1 - c, s], tidx_all)

    _zero_acc(zr_h, acc, r0)

    def cgrp(g, carry):
        k0 = g * CNT_GRP
        cps = [pltpu.async_copy(fbuf, acc.at[tidx_all.at[k0 + j]], s0,
                                add=True)
               for j in range(CNT_GRP)]
        for cp in cps:
            cp.wait()
        return carry

    lax.fori_loop(0, CHUNKS // CNT_GRP, cgrp, 0)
    _writeout(acc, cnt_o, c, r0)

    for tbl_h, out_h in ((xa_h, agg_a), (xb_h, agg_b)):
        _zero_acc(zr_h, acc, r0)
        for seg in range(SEGS):
            pltpu.sync_copy(ei_h.at[c, s, pl.ds(seg * SEGCH, SEGCH)], fseg)
            _pipe_pass(tbl_h, fseg, tidx_all, SEGCH, seg * SEGCH, rows, acc,
                       sems, fbuf)
        _writeout(acc, out_h, c, r0)


def _run_k1(xa, xb, ei, zrows, ones_rows):
    k1 = functools.partial(
        pl.kernel,
        out_type=[
            jax.ShapeDtypeStruct((NC, NP, HALF), F32),
            jax.ShapeDtypeStruct((NC, NP, HALF), F32),
            jax.ShapeDtypeStruct((NC, NP, HALF), F32),
        ],
        mesh=_sc_mesh(),
        scratch_types=[
            pltpu.VMEM((SEGCH, CB), jnp.int32),
            pltpu.VMEM((CHUNKS, CB), jnp.int32),
            pltpu.VMEM((4, CB, HALF // 2), jnp.int32),
            pltpu.VMEM((CB, HALF), F32),
            pltpu.VMEM_SHARED((ACCR, HALF), F32),
            pltpu.SemaphoreType.DMA,
            pltpu.SemaphoreType.DMA,
            pltpu.SemaphoreType.DMA,
            pltpu.SemaphoreType.DMA,
        ],
        compiler_params=_UNTILED_NOLAYOUT,
    )(_k1_body)
    return k1(xa, xb, ei, zrows, ones_rows)


def _make_k3_body(frm):
    def body(ptbl_h, ei_h, zr_h, out_o, fseg, tseg, rows, acc,
             s0, s1, s2, s3):
        c = lax.axis_index("c")
        s = lax.axis_index("s")
        r0 = s * APT

        pltpu.sync_copy(ei_h.at[frm, s, pl.ds(c * SEGCH, SEGCH)], fseg)
        pltpu.sync_copy(ei_h.at[1 - frm, s, pl.ds(c * SEGCH, SEGCH)], tseg)
        _zero_acc(zr_h, acc, r0)
        _pipe_pass(ptbl_h, fseg, tseg, SEGCH, 0, rows, acc, (s0, s1, s2, s3))
        _writeout(acc, out_o, c, r0)

    return body


def _run_k3(ptbl, ei, zrows64, frm):
    k3 = functools.partial(
        pl.kernel,
        out_type=jax.ShapeDtypeStruct((NC, NP, DIM_OUT), F32),
        mesh=_sc_mesh(),
        scratch_types=[
            pltpu.VMEM((SEGCH, CB), jnp.int32),
            pltpu.VMEM((SEGCH, CB), jnp.int32),
            pltpu.VMEM((4, CB, DIM_OUT), F32),
            pltpu.VMEM_SHARED((ACCR, DIM_OUT), F32),
            pltpu.SemaphoreType.DMA,
            pltpu.SemaphoreType.DMA,
            pltpu.SemaphoreType.DMA,
            pltpu.SemaphoreType.DMA,
        ],
        compiler_params=_UNTILED,
    )(_make_k3_body(frm))
    return k3(ptbl, ei, zrows64)


def _k2_body(x_r, aa_r, ab_r, cnt_r,
             ws1, bs1, wsd1, bsd1, wds1, bds1,
             ws2, wsd2, wds2,
             pself_r, p0_r, p1_r):
    cd = jnp.maximum(cnt_r[0, :, 0:1], 1.0)
    cs = jnp.maximum(cnt_r[1, :, 0:1], 1.0)
    a_s2d = jnp.concatenate([aa_r[0], ab_r[0]], axis=1) / cd
    a_d2s = jnp.concatenate([aa_r[1], ab_r[1]], axis=1) / cs
    xv = x_r[...]
    h = (jnp.dot(xv, ws1[...], preferred_element_type=F32) + bs1[...]
         + (1.0 - ALPHA_MIX) * (jnp.dot(a_s2d, wsd1[...], preferred_element_type=F32) + bsd1[...])
         + ALPHA_MIX * (jnp.dot(a_d2s, wds1[...], preferred_element_type=F32) + bds1[...]))
    h = SELU_SCALE * jnp.where(h > 0, h, SELU_ALPHA * (jnp.exp(h) - 1.0))
    pself_r[...] = jnp.dot(h, ws2[...], preferred_element_type=F32)
    p0_r[...] = jnp.dot(h, wsd2[...], preferred_element_type=F32)
    p1_r[...] = jnp.dot(h, wds2[...], preferred_element_type=F32)


def _run_k2(xp, agg_a, agg_b, cnt, ws1, bs1, wsd1, bsd1, wds1, bds1, ws2, wsd2, wds2):
    R = 1024
    grid = (NP // R,)
    wspec = pl.BlockSpec((DIM_IN, DIM_HID), lambda i: (0, 0))
    bspec = pl.BlockSpec((1, DIM_HID), lambda i: (0, 0))
    w2spec = pl.BlockSpec((DIM_HID, DIM_OUT), lambda i: (0, 0))
    return pl.pallas_call(
        _k2_body,
        grid=grid,
        in_specs=[
            pl.BlockSpec((R, DIM_IN), lambda i: (i, 0)),
            pl.BlockSpec((NC, R, HALF), lambda i: (0, i, 0)),
            pl.BlockSpec((NC, R, HALF), lambda i: (0, i, 0)),
            pl.BlockSpec((NC, R, HALF), lambda i: (0, i, 0)),
            wspec, bspec, wspec, bspec, wspec, bspec,
            w2spec, w2spec, w2spec,
        ],
        out_specs=[
            pl.BlockSpec((R, DIM_OUT), lambda i: (i, 0)),
            pl.BlockSpec((R, DIM_OUT), lambda i: (i, 0)),
            pl.BlockSpec((R, DIM_OUT), lambda i: (i, 0)),
        ],
        out_shape=[
            jax.ShapeDtypeStruct((NP, DIM_OUT), F32),
            jax.ShapeDtypeStruct((NP, DIM_OUT), F32),
            jax.ShapeDtypeStruct((NP, DIM_OUT), F32),
        ],
    )(xp, agg_a, agg_b, cnt, ws1, bs1, wsd1, bsd1, wds1, bds1, ws2, wsd2, wds2)


def _k4_body(pself_r, agg2a_r, agg2b_r, cnt_r, bs2, bsd2, bds2, out_r):
    cd = jnp.maximum(cnt_r[0, :, 0:1], 1.0)
    cs = jnp.maximum(cnt_r[1, :, 0:1], 1.0)
    s2d = agg2a_r[0] + agg2a_r[1]
    d2s = agg2b_r[0] + agg2b_r[1]
    out_r[...] = (pself_r[...] + bs2[...]
                  + (1.0 - ALPHA_MIX) * (s2d / cd + bsd2[...])
                  + ALPHA_MIX * (d2s / cs + bds2[...]))


def _run_k4(pself, agg2a, agg2b, cnt, bs2, bsd2, bds2):
    R = 2000
    grid = (N_NODES // R,)
    bspec = pl.BlockSpec((1, DIM_OUT), lambda i: (0, 0))
    return pl.pallas_call(
        _k4_body,
        grid=grid,
        in_specs=[
            pl.BlockSpec((R, DIM_OUT), lambda i: (i, 0)),
            pl.BlockSpec((NC, R, DIM_OUT), lambda i: (0, i, 0)),
            pl.BlockSpec((NC, R, DIM_OUT), lambda i: (0, i, 0)),
            pl.BlockSpec((NC, R, HALF), lambda i: (0, i, 0)),
            bspec, bspec, bspec,
        ],
        out_specs=pl.BlockSpec((R, DIM_OUT), lambda i: (i, 0)),
        out_shape=jax.ShapeDtypeStruct((N_NODES, DIM_OUT), F32),
    )(pself, agg2a, agg2b, cnt, bs2, bsd2, bds2)


def kernel(x, edge_index, W_self1, b_self1, W_s2d1, b_s2d1, W_d2s1, b_d2s1,
           W_self2, b_self2, W_s2d2, b_s2d2, W_d2s2, b_d2s2):
    xp = jnp.zeros((NP, DIM_IN), F32).at[:N_NODES].set(x)
    x16 = xp.astype(jnp.bfloat16)
    xa = lax.bitcast_convert_type(
        x16[:, :HALF].reshape(NP, HALF // 2, 2), jnp.int32)
    xb = lax.bitcast_convert_type(
        x16[:, HALF:].reshape(NP, HALF // 2, 2), jnp.int32)
    pad = jnp.full((2, EP - N_EDGES), PAD_IDX, jnp.int32)
    ei = jnp.concatenate([edge_index.astype(jnp.int32), pad], axis=1)
    ei = ei.reshape(2, NS, CHUNKS, CB)
    zrows = jnp.zeros((APT, HALF), F32)
    zrows64 = jnp.zeros((APT, DIM_OUT), F32)
    ones_rows = jnp.ones((CB, HALF), F32)

    agg_a, agg_b, cnt = _run_k1(xa, xb, ei, zrows, ones_rows)

    pself, pdir0, pdir1 = _run_k2(
        xp, agg_a, agg_b, cnt,
        W_self1, b_self1.reshape(1, DIM_HID),
        W_s2d1[_PERM, :], b_s2d1.reshape(1, DIM_HID),
        W_d2s1[_PERM, :], b_d2s1.reshape(1, DIM_HID),
        W_self2, W_s2d2, W_d2s2,
    )

    agg2a = _run_k3(pdir0, ei, zrows64, 0)
    agg2b = _run_k3(pdir1, ei, zrows64, 1)

    out = _run_k4(
        pself, agg2a, agg2b, cnt,
        b_self2.reshape(1, DIM_OUT),
        b_s2d2.reshape(1, DIM_OUT),
        b_d2s2.reshape(1, DIM_OUT),
    )
    return out


# speedup vs baseline: 4.9134x; 1.0631x over previous
import functools

import jax
import jax.numpy as jnp
from jax import lax
from jax.experimental import pallas as pl
from jax.experimental.pallas import tpu as pltpu
from jax.experimental.pallas import tpu_sc as plsc

F32 = jnp.float32

N_NODES = 10000
DIM_IN = 256
DIM_HID = 512
DIM_OUT = 64
N_EDGES = 160000
ALPHA_MIX = 0.5

NC = 2
NS = 16

NP = 10240
EP = 163840
CB = 64
EDGES_PER_TILE = EP // NS
CHUNKS = EDGES_PER_TILE // CB
HALF = DIM_IN // 2
ACCR = 10112
APT = ACCR // NS
PAD_IDX = ACCR - 1
SEGS = 2
SEGCH = CHUNKS // SEGS
CNT_GRP = 8

SELU_SCALE = 1.0507009873554805
SELU_ALPHA = 1.6732632423543772

_UNTILED = pltpu.CompilerParams(use_tc_tiling_on_sc=False)
_UNTILED_NOLAYOUT = pltpu.CompilerParams(use_tc_tiling_on_sc=False,
                                         needs_layout_passes=False)

_PERM = tuple(
    128 * h + 32 * q + 2 * p + r
    for h in range(2) for q in range(4) for r in range(2) for p in range(16)
)


def _sc_mesh():
    return plsc.VectorSubcoreMesh(
        core_axis_name="c", subcore_axis_name="s", num_cores=NC, num_subcores=NS
    )


def _zero_acc(zr_h, acc, r0):
    pltpu.sync_copy(zr_h, acc.at[pl.ds(r0, APT)])
    plsc.subcore_barrier()


def _writeout(acc, out_h, c, r0):
    plsc.subcore_barrier()
    pltpu.sync_copy(acc.at[pl.ds(r0, APT)], out_h.at[c, pl.ds(r0, APT)])


def _unpack_rows(rows, j, fbuf, p):
    def crow(r, carry):
        for q in range(HALF // 32):
            v = rows[j, r, pl.ds(16 * q, 16)]
            lo = plsc.bitcast(lax.shift_left(v, 16), F32)
            hi = plsc.bitcast(lax.bitwise_and(v, jnp.int32(-65536)), F32)
            fbuf[p, r, pl.ds(32 * q, 16)] = lo
            fbuf[p, r, pl.ds(32 * q + 16, 16)] = hi
        return carry

    lax.fori_loop(0, CB, crow, 0)


def _pipe_pass(tbl_h, fidx2d, tidx2d, nch, toff, rows, acc, sems, fbuf=None,
               ssems=None):
    s0, s1, s2, s3 = sems
    pltpu.async_copy(tbl_h.at[fidx2d.at[0]], rows.at[0], s0)
    pltpu.async_copy(tbl_h.at[fidx2d.at[1]], rows.at[1], s1)
    pltpu.async_copy(tbl_h.at[fidx2d.at[2]], rows.at[2], s2)
    ng = nch // 4

    def grp(g, carry):
        k0 = 4 * g
        for j in range(4):
            nb = (j + 3) % 4
            if j == 0:
                pltpu.async_copy(tbl_h.at[fidx2d.at[k0 + 3]], rows.at[3], s3)
            else:
                @pl.when(g < ng - 1)
                def _(nk=k0 + j + 3, nb=nb, sn=sems[nb]):
                    pltpu.async_copy(tbl_h.at[fidx2d.at[nk]], rows.at[nb], sn)
            pltpu.make_async_copy(tbl_h.at[fidx2d.at[k0 + j]], rows.at[j],
                                  sems[j]).wait()
            if fbuf is None:
                pltpu.sync_copy(rows.at[j], acc.at[tidx2d.at[toff + k0 + j]],
                                add=True)
            else:
                p = j % 2
                drain = pltpu.make_async_copy(
                    fbuf.at[p], acc.at[tidx2d.at[toff + k0 + j]], ssems[p])
                if j < 2:
                    @pl.when(g > 0)
                    def _(drain=drain):
                        drain.wait()
                else:
                    drain.wait()
                _unpack_rows(rows, j, fbuf, p)
                pltpu.async_copy(fbuf.at[p],
                                 acc.at[tidx2d.at[toff + k0 + j]], ssems[p],
                                 add=True)
        return carry

    lax.fori_loop(0, ng, grp, 0)
    if fbuf is not None:
        pltpu.make_async_copy(fbuf.at[0], acc.at[tidx2d.at[toff + nch - 2]],
                              ssems[0]).wait()
        pltpu.make_async_copy(fbuf.at[1], acc.at[tidx2d.at[toff + nch - 1]],
                              ssems[1]).wait()


def _k1_body(xa_h, xb_h, ei_h, zr_h, on_h,
             agg_a, agg_b, cnt_o,
             fseg, tidx_all, rows, fbuf, acc, s0, s1, s2, s3, s4, s5):
    c = lax.axis_index("c")
    s = lax.axis_index("s")
    r0 = s * APT
    sems = (s0, s1, s2, s3)
    ssems = (s4, s5)

    pltpu.sync_copy(on_h, fbuf.at[0])
    pltpu.sync_copy(ei_h.at[1 - c, s], tidx_all)

    _zero_acc(zr_h, acc, r0)

    def cgrp(g, carry):
        k0 = g * CNT_GRP
        cps = [pltpu.async_copy(fbuf.at[0], acc.at[tidx_all.at[k0 + j]], s0,
                                add=True)
               for j in range(CNT_GRP)]
        for cp in cps:
            cp.wait()
        return carry

    lax.fori_loop(0, CHUNKS // CNT_GRP, cgrp, 0)
    _writeout(acc, cnt_o, c, r0)

    for tbl_h, out_h in ((xa_h, agg_a), (xb_h, agg_b)):
        _zero_acc(zr_h, acc, r0)
        for seg in range(SEGS):
            pltpu.sync_copy(ei_h.at[c, s, pl.ds(seg * SEGCH, SEGCH)], fseg)
            _pipe_pass(tbl_h, fseg, tidx_all, SEGCH, seg * SEGCH, rows, acc,
                       sems, fbuf, ssems)
        _writeout(acc, out_h, c, r0)


def _run_k1(xa, xb, ei, zrows, ones_rows):
    k1 = functools.partial(
        pl.kernel,
        out_type=[
            jax.ShapeDtypeStruct((NC, NP, HALF), F32),
            jax.ShapeDtypeStruct((NC, NP, HALF), F32),
            jax.ShapeDtypeStruct((NC, NP, HALF), F32),
        ],
        mesh=_sc_mesh(),
        scratch_types=[
            pltpu.VMEM((SEGCH, CB), jnp.int32),
            pltpu.VMEM((CHUNKS, CB), jnp.int32),
            pltpu.VMEM((4, CB, HALF // 2), jnp.int32),
            pltpu.VMEM((2, CB, HALF), F32),
            pltpu.VMEM_SHARED((ACCR, HALF), F32),
            pltpu.SemaphoreType.DMA,
            pltpu.SemaphoreType.DMA,
            pltpu.SemaphoreType.DMA,
            pltpu.SemaphoreType.DMA,
            pltpu.SemaphoreType.DMA,
            pltpu.SemaphoreType.DMA,
        ],
        compiler_params=_UNTILED_NOLAYOUT,
    )(_k1_body)
    return k1(xa, xb, ei, zrows, ones_rows)


def _make_k3_body(frm):
    def body(ptbl_h, ei_h, zr_h, out_o, fseg, tseg, rows, acc,
             s0, s1, s2, s3):
        c = lax.axis_index("c")
        s = lax.axis_index("s")
        r0 = s * APT

        pltpu.sync_copy(ei_h.at[frm, s, pl.ds(c * SEGCH, SEGCH)], fseg)
        pltpu.sync_copy(ei_h.at[1 - frm, s, pl.ds(c * SEGCH, SEGCH)], tseg)
        _zero_acc(zr_h, acc, r0)
        _pipe_pass(ptbl_h, fseg, tseg, SEGCH, 0, rows, acc, (s0, s1, s2, s3))
        _writeout(acc, out_o, c, r0)

    return body


def _run_k3(ptbl, ei, zrows64, frm):
    k3 = functools.partial(
        pl.kernel,
        out_type=jax.ShapeDtypeStruct((NC, NP, DIM_OUT), F32),
        mesh=_sc_mesh(),
        scratch_types=[
            pltpu.VMEM((SEGCH, CB), jnp.int32),
            pltpu.VMEM((SEGCH, CB), jnp.int32),
            pltpu.VMEM((4, CB, DIM_OUT), F32),
            pltpu.VMEM_SHARED((ACCR, DIM_OUT), F32),
            pltpu.SemaphoreType.DMA,
            pltpu.SemaphoreType.DMA,
            pltpu.SemaphoreType.DMA,
            pltpu.SemaphoreType.DMA,
        ],
        compiler_params=_UNTILED,
    )(_make_k3_body(frm))
    return k3(ptbl, ei, zrows64)


def _k2_body(x_r, aa_r, ab_r, cnt_r,
             ws1, bs1, wsd1, bsd1, wds1, bds1,
             ws2, wsd2, wds2,
             pself_r, p0_r, p1_r):
    cd = jnp.maximum(cnt_r[0, :, 0:1], 1.0)
    cs = jnp.maximum(cnt_r[1, :, 0:1], 1.0)
    a_s2d = jnp.concatenate([aa_r[0], ab_r[0]], axis=1) / cd
    a_d2s = jnp.concatenate([aa_r[1], ab_r[1]], axis=1) / cs
    xv = x_r[...]
    h = (jnp.dot(xv, ws1[...], preferred_element_type=F32) + bs1[...]
         + (1.0 - ALPHA_MIX) * (jnp.dot(a_s2d, wsd1[...], preferred_element_type=F32) + bsd1[...])
         + ALPHA_MIX * (jnp.dot(a_d2s, wds1[...], preferred_element_type=F32) + bds1[...]))
    h = SELU_SCALE * jnp.where(h > 0, h, SELU_ALPHA * (jnp.exp(h) - 1.0))
    pself_r[...] = jnp.dot(h, ws2[...], preferred_element_type=F32)
    p0_r[...] = jnp.dot(h, wsd2[...], preferred_element_type=F32)
    p1_r[...] = jnp.dot(h, wds2[...], preferred_element_type=F32)


def _run_k2(xp, agg_a, agg_b, cnt, ws1, bs1, wsd1, bsd1, wds1, bds1, ws2, wsd2, wds2):
    R = 1024
    grid = (NP // R,)
    wspec = pl.BlockSpec((DIM_IN, DIM_HID), lambda i: (0, 0))
    bspec = pl.BlockSpec((1, DIM_HID), lambda i: (0, 0))
    w2spec = pl.BlockSpec((DIM_HID, DIM_OUT), lambda i: (0, 0))
    return pl.pallas_call(
        _k2_body,
        grid=grid,
        in_specs=[
            pl.BlockSpec((R, DIM_IN), lambda i: (i, 0)),
            pl.BlockSpec((NC, R, HALF), lambda i: (0, i, 0)),
            pl.BlockSpec((NC, R, HALF), lambda i: (0, i, 0)),
            pl.BlockSpec((NC, R, HALF), lambda i: (0, i, 0)),
            wspec, bspec, wspec, bspec, wspec, bspec,
            w2spec, w2spec, w2spec,
        ],
        out_specs=[
            pl.BlockSpec((R, DIM_OUT), lambda i: (i, 0)),
            pl.BlockSpec((R, DIM_OUT), lambda i: (i, 0)),
            pl.BlockSpec((R, DIM_OUT), lambda i: (i, 0)),
        ],
        out_shape=[
            jax.ShapeDtypeStruct((NP, DIM_OUT), F32),
            jax.ShapeDtypeStruct((NP, DIM_OUT), F32),
            jax.ShapeDtypeStruct((NP, DIM_OUT), F32),
        ],
    )(xp, agg_a, agg_b, cnt, ws1, bs1, wsd1, bsd1, wds1, bds1, ws2, wsd2, wds2)


def _k4_body(pself_r, agg2a_r, agg2b_r, cnt_r, bs2, bsd2, bds2, out_r):
    cd = jnp.maximum(cnt_r[0, :, 0:1], 1.0)
    cs = jnp.maximum(cnt_r[1, :, 0:1], 1.0)
    s2d = agg2a_r[0] + agg2a_r[1]
    d2s = agg2b_r[0] + agg2b_r[1]
    out_r[...] = (pself_r[...] + bs2[...]
                  + (1.0 - ALPHA_MIX) * (s2d / cd + bsd2[...])
                  + ALPHA_MIX * (d2s / cs + bds2[...]))


def _run_k4(pself, agg2a, agg2b, cnt, bs2, bsd2, bds2):
    R = 2000
    grid = (N_NODES // R,)
    bspec = pl.BlockSpec((1, DIM_OUT), lambda i: (0, 0))
    return pl.pallas_call(
        _k4_body,
        grid=grid,
        in_specs=[
            pl.BlockSpec((R, DIM_OUT), lambda i: (i, 0)),
            pl.BlockSpec((NC, R, DIM_OUT), lambda i: (0, i, 0)),
            pl.BlockSpec((NC, R, DIM_OUT), lambda i: (0, i, 0)),
            pl.BlockSpec((NC, R, HALF), lambda i: (0, i, 0)),
            bspec, bspec, bspec,
        ],
        out_specs=pl.BlockSpec((R, DIM_OUT), lambda i: (i, 0)),
        out_shape=jax.ShapeDtypeStruct((N_NODES, DIM_OUT), F32),
    )(pself, agg2a, agg2b, cnt, bs2, bsd2, bds2)


def kernel(x, edge_index, W_self1, b_self1, W_s2d1, b_s2d1, W_d2s1, b_d2s1,
           W_self2, b_self2, W_s2d2, b_s2d2, W_d2s2, b_d2s2):
    xp = jnp.zeros((NP, DIM_IN), F32).at[:N_NODES].set(x)
    x16 = xp.astype(jnp.bfloat16)
    xa = lax.bitcast_convert_type(
        x16[:, :HALF].reshape(NP, HALF // 2, 2), jnp.int32)
    xb = lax.bitcast_convert_type(
        x16[:, HALF:].reshape(NP, HALF // 2, 2), jnp.int32)
    pad = jnp.full((2, EP - N_EDGES), PAD_IDX, jnp.int32)
    ei = jnp.concatenate([edge_index.astype(jnp.int32), pad], axis=1)
    ei = ei.reshape(2, NS, CHUNKS, CB)
    zrows = jnp.zeros((APT, HALF), F32)
    zrows64 = jnp.zeros((APT, DIM_OUT), F32)
    ones_rows = jnp.ones((CB, HALF), F32)

    agg_a, agg_b, cnt = _run_k1(xa, xb, ei, zrows, ones_rows)

    pself, pdir0, pdir1 = _run_k2(
        xp, agg_a, agg_b, cnt,
        W_self1, b_self1.reshape(1, DIM_HID),
        W_s2d1[_PERM, :], b_s2d1.reshape(1, DIM_HID),
        W_d2s1[_PERM, :], b_d2s1.reshape(1, DIM_HID),
        W_self2, W_s2d2, W_d2s2,
    )

    agg2a = _run_k3(pdir0, ei, zrows64, 0)
    agg2b = _run_k3(pdir1, ei, zrows64, 1)

    out = _run_k4(
        pself, agg2a, agg2b, cnt,
        b_self2.reshape(1, DIM_OUT),
        b_s2d2.reshape(1, DIM_OUT),
        b_d2s2.reshape(1, DIM_OUT),
    )
    return out
